# baseline (speedup 1.0000x reference)
import functools

import jax
import jax.numpy as jnp
from jax.experimental import pallas as pl
from jax.experimental.pallas import tpu as pltpu

_ALPHA = 3.0
_K = 32
_RBLK = 128
_ONE_BITS = 0x3F800000


def _nv_kernel(emb_ref, wt_ref, b_ref, out_ref):
    y = jnp.dot(emb_ref[...], wt_ref[...], preferred_element_type=jnp.float32)
    out_ref[...] = jnp.tanh(_ALPHA * (y + b_ref[...]))


def _exact_dot(x, y):
    return jax.lax.dot_general(
        x, y, (((1,), (0,)), ((), ())),
        precision=jax.lax.Precision.HIGHEST,
        preferred_element_type=jnp.float32)


def _adj_kernel(nv_ref, nvt_ref, ut_ref, st_ref, gmap_ref, gmapt_ref, out_ref, *,
                n_cols):
    a = jnp.dot(nv_ref[...], nvt_ref[...], preferred_element_type=jnp.float32)
    v = jnp.maximum(jnp.tanh(_ALPHA * a), 0.0)
    col = jax.lax.broadcasted_iota(jnp.int32, v.shape, 1)
    v = jnp.where(col < n_cols, v, 0.0)
    u = jax.lax.bitcast_convert_type(v, jnp.int32)

    rows = v.shape[0]
    ncols_pad = v.shape[1]
    ngrp = ncols_pad // 128
    ones_col = jnp.ones((ncols_pad, 1), jnp.float32)

    def body(_, carry):
        lo, hi = carry
        mid = (lo + hi + 1) >> 1
        cnt = jnp.sum((u >= mid).astype(jnp.int32), axis=1, keepdims=True)
        ok = cnt >= _K
        return jnp.where(ok, mid, lo), jnp.where(ok, hi, mid)

    def _full_search():
        lo0 = jnp.zeros((rows, 1), jnp.int32)
        hi0 = jnp.full((rows, 1), _ONE_BITS + 1, jnp.int32)
        tau_s, _ = jax.lax.fori_loop(0, 31, body, (lo0, hi0))
        return tau_s

    sat_f = (u >= _ONE_BITS).astype(jnp.float32)
    c1 = _exact_dot(sat_f, ones_col)
    all_sat = jnp.min(c1) >= float(_K)
    tau = jax.lax.cond(
        all_sat, lambda: jnp.full((rows, 1), _ONE_BITS, jnp.int32),
        _full_search)

    gt = u > tau
    gt_f = gt.astype(jnp.float32)
    tie_f = (u == tau).astype(jnp.float32)
    cnt_gt = _exact_dot(gt_f, ones_col)
    need = float(_K) - cnt_gt

    pref = _exact_dot(tie_f.reshape(rows * ngrp, 128),
                      ut_ref[...]).reshape(rows, ncols_pad)
    grp = _exact_dot(tie_f, gmap_ref[...])
    offs = _exact_dot(grp, st_ref[...])
    offs_full = _exact_dot(offs, gmapt_ref[...])
    incl = pref + offs_full
    keep = gt | ((tie_f > 0.0) & (incl <= need))
    res = jnp.where(keep, v, 0.0)
    out_ref[...] = res[:, :n_cols]


def kernel(idx, emb1, W, b):
    n, d = emb1.shape
    x = jnp.take(emb1, idx, axis=0)
    npad = ((n + _RBLK - 1) // _RBLK) * _RBLK
    xp = jnp.pad(x, ((0, npad - n), (0, 0)))
    wt = W.T
    b2 = b.reshape(1, d)

    nv = pl.pallas_call(
        _nv_kernel,
        out_shape=jax.ShapeDtypeStruct((npad, d), jnp.float32),
    )(xp, wt, b2)
    nvt = nv.T

    ngrp = npad // 128
    li = jnp.arange(128, dtype=jnp.int32)
    ut = (li[:, None] <= li[None, :]).astype(jnp.float32)
    gi = jnp.arange(ngrp, dtype=jnp.int32)
    st = (gi[:, None] < gi[None, :]).astype(jnp.float32)
    ci = jnp.arange(npad, dtype=jnp.int32)
    gmapt = (gi[:, None] == (ci[None, :] // 128)).astype(jnp.float32)
    gmap = gmapt.T

    grid = npad // _RBLK
    adj = pl.pallas_call(
        functools.partial(_adj_kernel, n_cols=n),
        grid=(grid,),
        in_specs=[
            pl.BlockSpec((_RBLK, d), lambda i: (i, 0)),
            pl.BlockSpec((d, npad), lambda i: (0, 0)),
            pl.BlockSpec((128, 128), lambda i: (0, 0)),
            pl.BlockSpec((ngrp, ngrp), lambda i: (0, 0)),
            pl.BlockSpec((npad, ngrp), lambda i: (0, 0)),
            pl.BlockSpec((ngrp, npad), lambda i: (0, 0)),
        ],
        out_specs=pl.BlockSpec((_RBLK, n), lambda i: (i, 0)),
        out_shape=jax.ShapeDtypeStruct((n, n), jnp.float32),
        compiler_params=pltpu.CompilerParams(
            dimension_semantics=("parallel",)
        ),
    )(nv, nvt, ut, st, gmap, gmapt)
    return adj
